# baseline (speedup 1.0000x reference)
import functools

import jax
import jax.numpy as jnp
from jax import lax
from jax.experimental import pallas as pl
from jax.experimental.pallas import tpu as pltpu
from jax.experimental.pallas import tpu_sc as plsc

_N = 4096
_T = 20
_E = 128
_H = 128
_O = 32


def _gather_rows_sc(embed, idx):
    V, D = embed.shape
    (B,) = idx.shape
    info = plsc.get_sparse_core_info()
    nw = info.num_cores * info.num_subcores
    b_per_w = B // nw
    ch = 320
    nb = 3
    n_ch = b_per_w // ch
    mesh = plsc.VectorSubcoreMesh(core_axis_name="c", subcore_axis_name="s")

    @functools.partial(
        pl.kernel,
        out_type=jax.ShapeDtypeStruct((B, D), jnp.float32),
        mesh=mesh,
        scratch_types=[
            pltpu.VMEM((b_per_w,), jnp.int32),
            pltpu.VMEM((nb, ch, D), jnp.float32),
            [pltpu.SemaphoreType.DMA] * nb,
            [pltpu.SemaphoreType.DMA] * nb,
        ],
    )
    def k(table_hbm, idx_hbm, out_hbm, idx_v, rows_v, gsems, ssems):
        wid = lax.axis_index("s") * info.num_cores + lax.axis_index("c")
        base = wid * b_per_w
        pltpu.sync_copy(idx_hbm.at[pl.ds(base, b_per_w)], idx_v)

        def gather(i):
            return pltpu.async_copy(
                table_hbm.at[idx_v.at[pl.ds(i * ch, ch)]],
                rows_v.at[i % nb], gsems[i % nb],
            )

        gat = [None] * n_ch
        sca = [None] * n_ch
        for i in range(min(2, n_ch)):
            gat[i] = gather(i)
        for i in range(n_ch):
            b = i % nb
            gat[i].wait()
            sca[i] = pltpu.async_copy(
                rows_v.at[b], out_hbm.at[pl.ds(base + i * ch, ch)], ssems[b]
            )
            j = i + 2
            if j < n_ch:
                if sca[j - nb] is not None:
                    sca[j - nb].wait()
                gat[j] = gather(j)
        for i in range(max(0, n_ch - nb), n_ch):
            sca[i].wait()

    return k(embed, idx)


def _sig(v):
    return 0.5 * jnp.tanh(0.5 * v) + 0.5


def _lstm_step(x_ref, wc_ref, b_ref, z_ref, c_ref):
    z_ref[:, :_E] = x_ref[0].astype(jnp.bfloat16)
    gates = jnp.dot(z_ref[...], wc_ref[...], preferred_element_type=jnp.float32)
    gates = gates + b_ref[...]
    i = _sig(gates[:, 0 * _H:1 * _H])
    f = _sig(gates[:, 1 * _H:2 * _H])
    g = jnp.tanh(gates[:, 2 * _H:3 * _H])
    o = _sig(gates[:, 3 * _H:4 * _H])
    c = f * c_ref[...] + i * g
    h = o * jnp.tanh(c)
    c_ref[...] = c
    hb = h.astype(jnp.bfloat16)
    z_ref[:, _E:] = hb
    return hb


def _lstm_body(x_ref, wc_ref, b_ref, w1_ref, out_ref, z_ref, c_ref):
    t = pl.program_id(0)

    @pl.when(t == 0)
    def _():
        z_ref[:, _E:] = jnp.zeros_like(z_ref[:, _E:])
        c_ref[...] = jnp.zeros_like(c_ref)

    hb = _lstm_step(x_ref, wc_ref, b_ref, z_ref, c_ref)

    @pl.when(t == pl.num_programs(0) - 1)
    def _():
        out_ref[...] = jnp.dot(hb, w1_ref[...], preferred_element_type=jnp.float32)


def _lstm(x, wc, b, W1bf):
    return pl.pallas_call(
        _lstm_body,
        grid=(_T,),
        in_specs=[
            pl.BlockSpec((1, _N, _E), lambda t: (t, 0, 0)),
            pl.BlockSpec((_E + _H, 4 * _H), lambda t: (0, 0)),
            pl.BlockSpec((1, 4 * _H), lambda t: (0, 0)),
            pl.BlockSpec((_H, 2 * _H), lambda t: (0, 0)),
        ],
        out_specs=pl.BlockSpec((_N, 2 * _H), lambda t: (0, 0)),
        out_shape=jax.ShapeDtypeStruct((_N, 2 * _H), jnp.float32),
        scratch_shapes=[
            pltpu.VMEM((_N, _E + _H), jnp.bfloat16),
            pltpu.VMEM((_N, _H), jnp.float32),
        ],
    )(x, wc, b, W1bf)


def _cast_body(a_ref, o_ref):
    o_ref[...] = a_ref[...].astype(jnp.bfloat16)


def _cast_adj(adj, bm):
    return pl.pallas_call(
        _cast_body,
        grid=(_N // bm,),
        in_specs=[pl.BlockSpec((bm, _N), lambda i: (i, 0))],
        out_specs=pl.BlockSpec((bm, _N), lambda i: (i, 0)),
        out_shape=jax.ShapeDtypeStruct((_N, _N), jnp.bfloat16),
    )(adj)


def _gcn1_body(adj_ref, s1_ref, w2_ref, b1_ref, out_ref):
    s = s1_ref[...].astype(jnp.bfloat16)
    t = jnp.dot(adj_ref[...], s, preferred_element_type=jnp.float32)
    t = jnp.maximum(t + b1_ref[...], 0.0)
    out_ref[...] = jnp.dot(t.astype(jnp.bfloat16), w2_ref[...],
                           preferred_element_type=jnp.float32)


def _gcn1(adj_bf, s1, W2bf, b1, bm):
    return pl.pallas_call(
        _gcn1_body,
        grid=(_N // bm,),
        in_specs=[
            pl.BlockSpec((bm, _N), lambda i: (i, 0)),
            pl.BlockSpec((_N, 2 * _H), lambda i: (0, 0)),
            pl.BlockSpec((2 * _H, _O), lambda i: (0, 0)),
            pl.BlockSpec((1, 2 * _H), lambda i: (0, 0)),
        ],
        out_specs=pl.BlockSpec((bm, _O), lambda i: (i, 0)),
        out_shape=jax.ShapeDtypeStruct((_N, _O), jnp.float32),
    )(adj_bf, s1, W2bf, b1)


def _gcn2_body(adj_ref, s2_ref, b2_ref, out_ref):
    s = s2_ref[...].astype(jnp.bfloat16)
    y = jnp.dot(adj_ref[...], s, preferred_element_type=jnp.float32)
    y = y + b2_ref[...]
    m = jnp.max(y, axis=1, keepdims=True)
    y = y - m
    lse = jnp.log(jnp.sum(jnp.exp(y), axis=1, keepdims=True))
    out_ref[...] = y - lse


def _gcn2(adj_bf, s2, b2, bm):
    return pl.pallas_call(
        _gcn2_body,
        grid=(_N // bm,),
        in_specs=[
            pl.BlockSpec((bm, _N), lambda i: (i, 0)),
            pl.BlockSpec((_N, _O), lambda i: (0, 0)),
            pl.BlockSpec((1, _O), lambda i: (0, 0)),
        ],
        out_specs=pl.BlockSpec((bm, _O), lambda i: (i, 0)),
        out_shape=jax.ShapeDtypeStruct((_N, _O), jnp.float32),
    )(adj_bf, s2, b2)


def kernel(inputs, adj, embed, W_ih, W_hh, b_ih, b_hh, W1, b1, W2, b2):
    adj_bf = _cast_adj(adj, 512)
    idx = jnp.transpose(inputs).reshape(-1).astype(jnp.int32)
    x = _gather_rows_sc(embed, idx).reshape(_T, _N, _E)

    b = (b_ih + b_hh).reshape(1, 4 * _H)
    wc = jnp.concatenate([W_ih.T, W_hh.T], axis=0).astype(jnp.bfloat16)
    support1 = _lstm(x, wc, b, W1.astype(jnp.bfloat16))
    support2 = _gcn1(adj_bf, support1, W2.astype(jnp.bfloat16),
                     b1.reshape(1, 2 * _H), 1024)
    return _gcn2(adj_bf, support2, b2.reshape(1, _O), 1024)


# speedup vs baseline: 1.2332x; 1.2332x over previous
import functools

import jax
import jax.numpy as jnp
from jax import lax
from jax.experimental import pallas as pl
from jax.experimental.pallas import tpu as pltpu
from jax.experimental.pallas import tpu_sc as plsc

_N = 4096
_T = 20
_E = 128
_H = 128
_O = 32


def _gather_rows_sc(embed, idx):
    V, D = embed.shape
    (B,) = idx.shape
    info = plsc.get_sparse_core_info()
    nw = info.num_cores * info.num_subcores
    b_per_w = B // nw
    ch = 320
    nb = 3
    n_ch = b_per_w // ch
    mesh = plsc.VectorSubcoreMesh(core_axis_name="c", subcore_axis_name="s")

    @functools.partial(
        pl.kernel,
        out_type=jax.ShapeDtypeStruct((B, D), jnp.float32),
        mesh=mesh,
        scratch_types=[
            pltpu.VMEM((b_per_w,), jnp.int32),
            pltpu.VMEM((nb, ch, D), jnp.float32),
            [pltpu.SemaphoreType.DMA] * nb,
            [pltpu.SemaphoreType.DMA] * nb,
        ],
    )
    def k(table_hbm, idx_hbm, out_hbm, idx_v, rows_v, gsems, ssems):
        wid = lax.axis_index("s") * info.num_cores + lax.axis_index("c")
        base = wid * b_per_w
        pltpu.sync_copy(idx_hbm.at[pl.ds(base, b_per_w)], idx_v)

        def gather(i):
            return pltpu.async_copy(
                table_hbm.at[idx_v.at[pl.ds(i * ch, ch)]],
                rows_v.at[i % nb], gsems[i % nb],
            )

        gat = [None] * n_ch
        sca = [None] * n_ch
        for i in range(min(2, n_ch)):
            gat[i] = gather(i)
        for i in range(n_ch):
            b = i % nb
            gat[i].wait()
            sca[i] = pltpu.async_copy(
                rows_v.at[b], out_hbm.at[pl.ds(base + i * ch, ch)], ssems[b]
            )
            j = i + 2
            if j < n_ch:
                if sca[j - nb] is not None:
                    sca[j - nb].wait()
                gat[j] = gather(j)
        for i in range(max(0, n_ch - nb), n_ch):
            sca[i].wait()

    return k(embed, idx)


def _sig(v):
    return 0.5 * jnp.tanh(0.5 * v) + 0.5


def _lstm_step(x_ref, wc_ref, b_ref, z_ref, c_ref):
    z_ref[:, :_E] = x_ref[0].astype(jnp.bfloat16)
    gates = jnp.dot(z_ref[...], wc_ref[...], preferred_element_type=jnp.float32)
    gates = gates + b_ref[...]
    i = _sig(gates[:, 0 * _H:1 * _H])
    f = _sig(gates[:, 1 * _H:2 * _H])
    g = jnp.tanh(gates[:, 2 * _H:3 * _H])
    o = _sig(gates[:, 3 * _H:4 * _H])
    c = f * c_ref[...] + i * g
    h = o * jnp.tanh(c)
    c_ref[...] = c
    hb = h.astype(jnp.bfloat16)
    z_ref[:, _E:] = hb
    return hb


_BR = _N // 16
_NBUF = 6


def _fused_body(x_ref, wc_ref, b_ref, w1_ref, w2_ref, b1_ref, b2_ref,
                adj_hbm, out_ref, z_ref, c_ref, s1_ref, s2_ref, abuf_ref,
                sems):
    t = pl.program_id(0)

    def _copy(m):
        blk = jnp.where(m < 16, m, m - 16)
        return pltpu.make_async_copy(
            adj_hbm.at[pl.ds(blk * _BR, _BR)],
            abuf_ref.at[m % _NBUF],
            sems.at[m % _NBUF],
        )

    @pl.when(t < _T)
    def _phase_a():
        @pl.when(t == 0)
        def _():
            z_ref[:, _E:] = jnp.zeros_like(z_ref[:, _E:])
            c_ref[...] = jnp.zeros_like(c_ref)

        hb = _lstm_step(x_ref, wc_ref, b_ref, z_ref, c_ref)

        @pl.when(t >= _T - _NBUF)
        def _():
            _copy(t - (_T - _NBUF)).start()

        @pl.when(t == _T - 1)
        def _():
            s1_ref[...] = jnp.dot(hb, w1_ref[...],
                                  preferred_element_type=jnp.float32)

    @pl.when((t >= _T) & (t < _T + 16))
    def _phase_b():
        u = t - _T
        _copy(u).wait()
        a = abuf_ref[u % _NBUF]
        t1 = jnp.dot(a, s1_ref[...], preferred_element_type=jnp.float32)
        t1 = jnp.maximum(t1 + b1_ref[...], 0.0)
        s2_ref[pl.ds(u * _BR, _BR), :] = jnp.dot(
            t1, w2_ref[...], preferred_element_type=jnp.float32)

        @pl.when(u + _NBUF < 16)
        def _():
            _copy(u + _NBUF).start()

    @pl.when(t >= _T + 16)
    def _phase_c():
        v = t - (_T + 16)
        m = 10 + v

        @pl.when(v >= _NBUF)
        def _():
            _copy(m).wait()

        a = abuf_ref[m % _NBUF]
        y = jnp.dot(a, s2_ref[...], preferred_element_type=jnp.float32)
        y = y + b2_ref[...]
        mx = jnp.max(y, axis=1, keepdims=True)
        y = y - mx
        lse = jnp.log(jnp.sum(jnp.exp(y), axis=1, keepdims=True))
        out_ref[...] = y - lse

        @pl.when(v <= 9)
        def _():
            _copy(16 + v).start()


def _fused_net(x, wc, b, W1bf, W2, b1, b2, adj):
    grid = _T + 32
    return pl.pallas_call(
        _fused_body,
        grid=(grid,),
        in_specs=[
            pl.BlockSpec((1, _N, _E), lambda t: (jnp.minimum(t, _T - 1), 0, 0)),
            pl.BlockSpec((_E + _H, 4 * _H), lambda t: (0, 0)),
            pl.BlockSpec((1, 4 * _H), lambda t: (0, 0)),
            pl.BlockSpec((_H, 2 * _H), lambda t: (0, 0)),
            pl.BlockSpec((2 * _H, _O), lambda t: (0, 0)),
            pl.BlockSpec((1, 2 * _H), lambda t: (0, 0)),
            pl.BlockSpec((1, _O), lambda t: (0, 0)),
            pl.BlockSpec(memory_space=pl.ANY),
        ],
        out_specs=pl.BlockSpec(
            (_BR, _O),
            lambda t: (jnp.where(t < _T + 16, 10, (t - (_T + 16) + 10) % 16), 0),
        ),
        out_shape=jax.ShapeDtypeStruct((_N, _O), jnp.float32),
        scratch_shapes=[
            pltpu.VMEM((_N, _E + _H), jnp.bfloat16),
            pltpu.VMEM((_N, _H), jnp.float32),
            pltpu.VMEM((_N, 2 * _H), jnp.float32),
            pltpu.VMEM((_N, _O), jnp.float32),
            pltpu.VMEM((_NBUF, _BR, _N), jnp.float32),
            pltpu.SemaphoreType.DMA((_NBUF,)),
        ],
    )(x, wc, b, W1bf, W2, b1, b2, adj)


def kernel(inputs, adj, embed, W_ih, W_hh, b_ih, b_hh, W1, b1, W2, b2):
    idx = jnp.transpose(inputs).reshape(-1).astype(jnp.int32)
    x = _gather_rows_sc(embed, idx).reshape(_T, _N, _E)

    b = (b_ih + b_hh).reshape(1, 4 * _H)
    wc = jnp.concatenate([W_ih.T, W_hh.T], axis=0).astype(jnp.bfloat16)
    return _fused_net(x, wc, b, W1.astype(jnp.bfloat16), W2,
                      b1.reshape(1, 2 * _H), b2.reshape(1, _O), adj)


# speedup vs baseline: 1.2895x; 1.0456x over previous
import functools

import jax
import jax.numpy as jnp
from jax import lax
from jax.experimental import pallas as pl
from jax.experimental.pallas import tpu as pltpu
from jax.experimental.pallas import tpu_sc as plsc

_N = 4096
_T = 20
_E = 128
_H = 128
_O = 32


def _gather_rows_sc(embed, idx):
    V, D = embed.shape
    (B,) = idx.shape
    info = plsc.get_sparse_core_info()
    nw = info.num_cores * info.num_subcores
    b_per_w = B // nw
    ch = 320
    nb = 3
    n_ch = b_per_w // ch
    mesh = plsc.VectorSubcoreMesh(core_axis_name="c", subcore_axis_name="s")

    @functools.partial(
        pl.kernel,
        out_type=jax.ShapeDtypeStruct((B, D), jnp.float32),
        mesh=mesh,
        scratch_types=[
            pltpu.VMEM((b_per_w,), jnp.int32),
            pltpu.VMEM((nb, ch, D), jnp.float32),
            [pltpu.SemaphoreType.DMA] * nb,
            [pltpu.SemaphoreType.DMA] * nb,
        ],
    )
    def k(table_hbm, idx_hbm, out_hbm, idx_v, rows_v, gsems, ssems):
        wid = lax.axis_index("s") * info.num_cores + lax.axis_index("c")
        base = wid * b_per_w
        pltpu.sync_copy(idx_hbm.at[pl.ds(base, b_per_w)], idx_v)

        def gather(i):
            return pltpu.async_copy(
                table_hbm.at[idx_v.at[pl.ds(i * ch, ch)]],
                rows_v.at[i % nb], gsems[i % nb],
            )

        gat = [None] * n_ch
        sca = [None] * n_ch
        for i in range(min(2, n_ch)):
            gat[i] = gather(i)
        for i in range(n_ch):
            b = i % nb
            gat[i].wait()
            sca[i] = pltpu.async_copy(
                rows_v.at[b], out_hbm.at[pl.ds(base + i * ch, ch)], ssems[b]
            )
            j = i + 2
            if j < n_ch:
                if sca[j - nb] is not None:
                    sca[j - nb].wait()
                gat[j] = gather(j)
        for i in range(max(0, n_ch - nb), n_ch):
            sca[i].wait()

    return k(embed, idx)


def _sig(v):
    return 0.5 * jnp.tanh(0.5 * v) + 0.5


def _lstm_step(x_ref, wc_ref, b_ref, z_ref, c_ref):
    z_ref[:, :_E] = x_ref[0].astype(jnp.bfloat16)
    gates = jnp.dot(z_ref[...], wc_ref[...],
                    preferred_element_type=jnp.float32)
    gates = (gates + b_ref[...]).astype(jnp.bfloat16)
    i = _sig(gates[:, 0 * _H:1 * _H])
    f = _sig(gates[:, 1 * _H:2 * _H])
    g = jnp.tanh(gates[:, 2 * _H:3 * _H])
    o = _sig(gates[:, 3 * _H:4 * _H])
    c = f * c_ref[...] + i * g
    h = o * jnp.tanh(c)
    c_ref[...] = c
    z_ref[:, _E:] = h
    return h


_BR = _N // 16
_NBUF = 6


def _fused_body(x_ref, wc_ref, b_ref, w1_ref, w2_ref, b1_ref, b2_ref,
                adj_hbm, out_ref, z_ref, c_ref, s1_ref, s2_ref, abuf_ref,
                sems):
    t = pl.program_id(0)

    def _copy(m):
        blk = jnp.where(m < 16, m, m - 16)
        return pltpu.make_async_copy(
            adj_hbm.at[pl.ds(blk * _BR, _BR)],
            abuf_ref.at[m % _NBUF],
            sems.at[m % _NBUF],
        )

    @pl.when(t < _T)
    def _phase_a():
        @pl.when(t == 0)
        def _():
            z_ref[:, _E:] = jnp.zeros_like(z_ref[:, _E:])
            c_ref[...] = jnp.zeros_like(c_ref)

        hb = _lstm_step(x_ref, wc_ref, b_ref, z_ref, c_ref)

        @pl.when(t >= _T - _NBUF)
        def _():
            _copy(t - (_T - _NBUF)).start()

        @pl.when(t == _T - 1)
        def _():
            s1_ref[...] = jnp.dot(hb, w1_ref[...],
                                  preferred_element_type=jnp.float32)

    @pl.when((t >= _T) & (t < _T + 16))
    def _phase_b():
        u = t - _T
        _copy(u).wait()
        a = abuf_ref[u % _NBUF]
        t1 = jnp.dot(a, s1_ref[...], preferred_element_type=jnp.float32)
        t1 = jnp.maximum(t1 + b1_ref[...], 0.0)
        s2_ref[pl.ds(u * _BR, _BR), :] = jnp.dot(
            t1, w2_ref[...], preferred_element_type=jnp.float32)

        @pl.when(u + _NBUF < 16)
        def _():
            _copy(u + _NBUF).start()

    @pl.when(t >= _T + 16)
    def _phase_c():
        v = t - (_T + 16)
        m = 10 + v

        @pl.when(v >= _NBUF)
        def _():
            _copy(m).wait()

        a = abuf_ref[m % _NBUF]
        y = jnp.dot(a, s2_ref[...], preferred_element_type=jnp.float32)
        y = y + b2_ref[...]
        mx = jnp.max(y, axis=1, keepdims=True)
        y = y - mx
        lse = jnp.log(jnp.sum(jnp.exp(y), axis=1, keepdims=True))
        out_ref[...] = y - lse

        @pl.when(v <= 9)
        def _():
            _copy(16 + v).start()


def _fused_net(x, wc, b, W1bf, W2, b1, b2, adj):
    grid = _T + 32
    return pl.pallas_call(
        _fused_body,
        grid=(grid,),
        in_specs=[
            pl.BlockSpec((1, _N, _E), lambda t: (jnp.minimum(t, _T - 1), 0, 0)),
            pl.BlockSpec((_E + _H, 4 * _H), lambda t: (0, 0)),
            pl.BlockSpec((1, 4 * _H), lambda t: (0, 0)),
            pl.BlockSpec((_H, 2 * _H), lambda t: (0, 0)),
            pl.BlockSpec((2 * _H, _O), lambda t: (0, 0)),
            pl.BlockSpec((1, 2 * _H), lambda t: (0, 0)),
            pl.BlockSpec((1, _O), lambda t: (0, 0)),
            pl.BlockSpec(memory_space=pl.ANY),
        ],
        out_specs=pl.BlockSpec(
            (_BR, _O),
            lambda t: (jnp.where(t < _T + 16, 10, (t - (_T + 16) + 10) % 16), 0),
        ),
        out_shape=jax.ShapeDtypeStruct((_N, _O), jnp.float32),
        scratch_shapes=[
            pltpu.VMEM((_N, _E + _H), jnp.bfloat16),
            pltpu.VMEM((_N, _H), jnp.bfloat16),
            pltpu.VMEM((_N, 2 * _H), jnp.float32),
            pltpu.VMEM((_N, _O), jnp.float32),
            pltpu.VMEM((_NBUF, _BR, _N), jnp.float32),
            pltpu.SemaphoreType.DMA((_NBUF,)),
        ],
    )(x, wc, b, W1bf, W2, b1, b2, adj)


def kernel(inputs, adj, embed, W_ih, W_hh, b_ih, b_hh, W1, b1, W2, b2):
    idx = jnp.transpose(inputs).reshape(-1).astype(jnp.int32)
    x = _gather_rows_sc(embed, idx).reshape(_T, _N, _E)

    b = (b_ih + b_hh).reshape(1, 4 * _H).astype(jnp.bfloat16)
    wc = jnp.concatenate([W_ih.T, W_hh.T], axis=0).astype(jnp.bfloat16)
    return _fused_net(x, wc, b, W1.astype(jnp.bfloat16), W2,
                      b1.reshape(1, 2 * _H), b2.reshape(1, _O), adj)
